# baseline (speedup 1.0000x reference)
import functools

import jax
import jax.numpy as jnp
from jax import lax
from jax.experimental import pallas as pl
from jax.experimental.pallas import tpu as pltpu
from jax.experimental.pallas import tpu_sc as plsc

N_NODES_K = 10000
D_K = 128
ACC_ROWS = 10240
CHUNK = 256
NC, NS = 2, 16
NW = NC * NS


KDEPTH = 1


def _sc_aggregate(feature, edge_pairs, n_chunks_per_worker):
    mesh = plsc.VectorSubcoreMesh(core_axis_name="c", subcore_axis_name="s")

    n = n_chunks_per_worker

    @functools.partial(
        pl.kernel,
        out_type=jax.ShapeDtypeStruct((NC, N_NODES_K, D_K), jnp.float32),
        mesh=mesh,
        scratch_types=[
            pltpu.VMEM((CHUNK,), jnp.int32),
            pltpu.VMEM((CHUNK,), jnp.int32),
            pltpu.VMEM((CHUNK, D_K), jnp.float32),
            pltpu.VMEM_SHARED((ACC_ROWS, D_K), jnp.float32),
            pltpu.SemaphoreType.DMA((KDEPTH,)),
        ],
    )
    def k(feat_hbm, pairs_hbm, out_hbm, src_v, dst_v, rows_v, acc_s, sem_g):
        core = lax.axis_index("c")
        sid = lax.axis_index("s")
        wid = sid * NC + core
        base = wid * n

        @pl.loop(0, 128)
        def _(r):
            @pl.loop(0, D_K, step=16)
            def _(c0):
                rows_v[r, pl.ds(c0, 16)] = jnp.zeros((16,), jnp.float32)

        stripe = ACC_ROWS // NS
        @pl.loop(0, stripe, step=128)
        def _(z):
            pltpu.sync_copy(
                rows_v.at[pl.ds(0, 128)],
                acc_s.at[pl.ds(sid * stripe + z, 128)],
            )

        plsc.subcore_barrier()

        @pl.loop(0, n)
        def _(j):
            pltpu.sync_copy(pairs_hbm.at[base + j].at[0], src_v)
            pltpu.sync_copy(pairs_hbm.at[base + j].at[1], dst_v)
            pltpu.async_copy(
                feat_hbm.at[src_v], rows_v, sem_g.at[0]
            ).wait()
            pltpu.sync_copy(rows_v, acc_s.at[dst_v], add=True)

        plsc.subcore_barrier()

        @pl.when(sid < NS - 1)
        def _():
            pltpu.sync_copy(
                acc_s.at[pl.ds(sid * stripe, stripe)],
                out_hbm.at[core].at[pl.ds(sid * stripe, stripe)],
            )

        @pl.when(sid == NS - 1)
        def _():
            last = N_NODES_K - (NS - 1) * stripe
            pltpu.sync_copy(
                acc_s.at[pl.ds((NS - 1) * stripe, last)],
                out_hbm.at[core].at[pl.ds((NS - 1) * stripe, last)],
            )

    return k(feature, edge_pairs)


def _tc_body(p_ref, w_ref, b_ref, o_ref):
    agg = p_ref[0] + p_ref[1]
    h = jnp.dot(agg, w_ref[...], preferred_element_type=jnp.float32)
    o_ref[...] = jnp.maximum(h + b_ref[...], 0.0)


def _tc_apply(partials, W, b):
    blk = 2000
    return pl.pallas_call(
        _tc_body,
        grid=(N_NODES_K // blk,),
        in_specs=[
            pl.BlockSpec((NC, blk, D_K), lambda i: (0, i, 0)),
            pl.BlockSpec((D_K, D_K), lambda i: (0, 0)),
            pl.BlockSpec((1, D_K), lambda i: (0, 0)),
        ],
        out_specs=pl.BlockSpec((blk, D_K), lambda i: (i, 0)),
        out_shape=jax.ShapeDtypeStruct((N_NODES_K, D_K), jnp.float32),
    )(partials, W, b.reshape(1, D_K))


def kernel(feature, edge_index, W, b):
    e = edge_index.shape[1]
    quantum = NW * KDEPTH * CHUNK
    epad = ((e + quantum - 1) // quantum) * quantum
    pad = epad - e
    src = jnp.concatenate([edge_index[0], jnp.zeros((pad,), jnp.int32)])
    dst = jnp.concatenate(
        [edge_index[1], jnp.full((pad,), N_NODES_K, jnp.int32)]
    )
    pairs = jnp.stack(
        [src.reshape(-1, CHUNK), dst.reshape(-1, CHUNK)], axis=1
    )
    partials = _sc_aggregate(feature, pairs, epad // (NW * CHUNK))
    return _tc_apply(partials, W, b)


# speedup vs baseline: 1.3531x; 1.3531x over previous
import functools

import jax
import jax.numpy as jnp
from jax import lax
from jax.experimental import pallas as pl
from jax.experimental.pallas import tpu as pltpu
from jax.experimental.pallas import tpu_sc as plsc

N_NODES_K = 10000
D_K = 128
ACC_ROWS = 10240
CHUNK = 128
NC, NS = 2, 16
NW = NC * NS


KDEPTH = 1


def _sc_aggregate(feature, edge_pairs, n_chunks_per_worker):
    mesh = plsc.VectorSubcoreMesh(core_axis_name="c", subcore_axis_name="s")

    n = n_chunks_per_worker

    @functools.partial(
        pl.kernel,
        out_type=jax.ShapeDtypeStruct((NC, N_NODES_K, D_K), jnp.float32),
        mesh=mesh,
        scratch_types=[
            pltpu.VMEM((CHUNK,), jnp.int32),
            pltpu.VMEM((CHUNK,), jnp.int32),
            pltpu.VMEM((CHUNK, D_K), jnp.float32),
            pltpu.VMEM_SHARED((ACC_ROWS, D_K), jnp.float32),
            pltpu.SemaphoreType.DMA((KDEPTH,)),
        ],
    )
    def k(feat_hbm, pairs_hbm, out_hbm, src_v, dst_v, rows_v, acc_s, sem_g):
        core = lax.axis_index("c")
        sid = lax.axis_index("s")
        wid = sid * NC + core
        base = wid * n

        @pl.loop(0, 128)
        def _(r):
            @pl.loop(0, D_K, step=16)
            def _(c0):
                rows_v[r, pl.ds(c0, 16)] = jnp.zeros((16,), jnp.float32)

        stripe = ACC_ROWS // NS
        @pl.loop(0, stripe, step=128)
        def _(z):
            pltpu.sync_copy(
                rows_v.at[pl.ds(0, 128)],
                acc_s.at[pl.ds(sid * stripe + z, 128)],
            )

        plsc.subcore_barrier()

        @pl.loop(0, n)
        def _(j):
            pltpu.sync_copy(pairs_hbm.at[base + j].at[0], src_v)
            pltpu.sync_copy(pairs_hbm.at[base + j].at[1], dst_v)
            pltpu.async_copy(
                feat_hbm.at[src_v], rows_v, sem_g.at[0]
            ).wait()
            pltpu.sync_copy(rows_v, acc_s.at[dst_v], add=True)

        plsc.subcore_barrier()

        @pl.when(sid < NS - 1)
        def _():
            pltpu.sync_copy(
                acc_s.at[pl.ds(sid * stripe, stripe)],
                out_hbm.at[core].at[pl.ds(sid * stripe, stripe)],
            )

        @pl.when(sid == NS - 1)
        def _():
            last = N_NODES_K - (NS - 1) * stripe
            pltpu.sync_copy(
                acc_s.at[pl.ds((NS - 1) * stripe, last)],
                out_hbm.at[core].at[pl.ds((NS - 1) * stripe, last)],
            )

    return k(feature, edge_pairs)


def _tc_body(p_ref, w_ref, b_ref, o_ref):
    agg = p_ref[0] + p_ref[1]
    h = jnp.dot(agg, w_ref[...], preferred_element_type=jnp.float32)
    o_ref[...] = jnp.maximum(h + b_ref[...], 0.0)


def _tc_apply(partials, W, b):
    blk = 2000
    return pl.pallas_call(
        _tc_body,
        grid=(N_NODES_K // blk,),
        in_specs=[
            pl.BlockSpec((NC, blk, D_K), lambda i: (0, i, 0)),
            pl.BlockSpec((D_K, D_K), lambda i: (0, 0)),
            pl.BlockSpec((1, D_K), lambda i: (0, 0)),
        ],
        out_specs=pl.BlockSpec((blk, D_K), lambda i: (i, 0)),
        out_shape=jax.ShapeDtypeStruct((N_NODES_K, D_K), jnp.float32),
    )(partials, W, b.reshape(1, D_K))


def kernel(feature, edge_index, W, b):
    e = edge_index.shape[1]
    quantum = NW * KDEPTH * CHUNK
    epad = ((e + quantum - 1) // quantum) * quantum
    pad = epad - e
    src = jnp.concatenate([edge_index[0], jnp.zeros((pad,), jnp.int32)])
    dst = jnp.concatenate(
        [edge_index[1], jnp.full((pad,), N_NODES_K, jnp.int32)]
    )
    pairs = jnp.stack(
        [src.reshape(-1, CHUNK), dst.reshape(-1, CHUNK)], axis=1
    )
    partials = _sc_aggregate(feature, pairs, epad // (NW * CHUNK))
    return _tc_apply(partials, W, b)


# speedup vs baseline: 1.3547x; 1.0012x over previous
import functools

import jax
import jax.numpy as jnp
from jax import lax
from jax.experimental import pallas as pl
from jax.experimental.pallas import tpu as pltpu
from jax.experimental.pallas import tpu_sc as plsc

N_NODES_K = 10000
D_K = 128
ACC_ROWS = 10240
CHUNK = 128
NC, NS = 2, 16
NW = NC * NS


KDEPTH = 1


def _sc_aggregate(feature, edge_pairs, n_chunks_per_worker):
    mesh = plsc.VectorSubcoreMesh(core_axis_name="c", subcore_axis_name="s")

    n = n_chunks_per_worker

    @functools.partial(
        pl.kernel,
        out_type=jax.ShapeDtypeStruct((NC, N_NODES_K, D_K), jnp.float32),
        mesh=mesh,
        scratch_types=[
            pltpu.VMEM((CHUNK,), jnp.int32),
            pltpu.VMEM((CHUNK,), jnp.int32),
            pltpu.VMEM((CHUNK, D_K), jnp.float32),
            pltpu.VMEM_SHARED((ACC_ROWS, D_K), jnp.float32),
            pltpu.SemaphoreType.DMA((KDEPTH,)),
        ],
    )
    def k(feat_hbm, pairs_hbm, out_hbm, src_v, dst_v, rows_v, acc_s, sem_g):
        core = lax.axis_index("c")
        sid = lax.axis_index("s")
        wid = sid * NC + core
        base = wid * n

        @pl.loop(0, 128)
        def _(r):
            @pl.loop(0, D_K, step=16)
            def _(c0):
                rows_v[r, pl.ds(c0, 16)] = jnp.zeros((16,), jnp.float32)

        stripe = ACC_ROWS // NS
        @pl.loop(0, stripe, step=128)
        def _(z):
            pltpu.sync_copy(
                rows_v.at[pl.ds(0, 128)],
                acc_s.at[pl.ds(sid * stripe + z, 128)],
            )

        plsc.subcore_barrier()

        @pl.loop(0, n)
        def _(j):
            pltpu.sync_copy(pairs_hbm.at[base + j].at[0], src_v)
            pltpu.sync_copy(pairs_hbm.at[base + j].at[1], dst_v)
            pltpu.async_copy(
                feat_hbm.at[src_v], rows_v, sem_g.at[0]
            ).wait()
            pltpu.sync_copy(rows_v, acc_s.at[dst_v], add=True)

        plsc.subcore_barrier()

        @pl.when(sid < NS - 1)
        def _():
            pltpu.sync_copy(
                acc_s.at[pl.ds(sid * stripe, stripe)],
                out_hbm.at[core].at[pl.ds(sid * stripe, stripe)],
            )

        @pl.when(sid == NS - 1)
        def _():
            last = N_NODES_K - (NS - 1) * stripe
            pltpu.sync_copy(
                acc_s.at[pl.ds((NS - 1) * stripe, last)],
                out_hbm.at[core].at[pl.ds((NS - 1) * stripe, last)],
            )

    return k(feature, edge_pairs)


def _tc_body(p_ref, w_ref, b_ref, o_ref):
    agg = p_ref[0] + p_ref[1]
    h = jnp.dot(agg, w_ref[...], preferred_element_type=jnp.float32)
    o_ref[...] = jnp.maximum(h + b_ref[...], 0.0)


def _tc_apply(partials, W, b):
    blk = 2000
    return pl.pallas_call(
        _tc_body,
        grid=(N_NODES_K // blk,),
        in_specs=[
            pl.BlockSpec((NC, blk, D_K), lambda i: (0, i, 0)),
            pl.BlockSpec((D_K, D_K), lambda i: (0, 0)),
            pl.BlockSpec((1, D_K), lambda i: (0, 0)),
        ],
        out_specs=pl.BlockSpec((blk, D_K), lambda i: (i, 0)),
        out_shape=jax.ShapeDtypeStruct((N_NODES_K, D_K), jnp.float32),
    )(partials, W, b.reshape(1, D_K))


def kernel(feature, edge_index, W, b):
    e = edge_index.shape[1]
    quantum = NW * KDEPTH * CHUNK
    epad = ((e + quantum - 1) // quantum) * quantum
    pad = epad - e
    src = jnp.concatenate([edge_index[0], jnp.zeros((pad,), jnp.int32)])
    dummy = N_NODES_K + jnp.arange(pad, dtype=jnp.int32) % (
        ACC_ROWS - N_NODES_K
    )
    dst = jnp.concatenate([edge_index[1], dummy])
    pairs = jnp.stack(
        [src.reshape(-1, CHUNK), dst.reshape(-1, CHUNK)], axis=1
    )
    partials = _sc_aggregate(feature, pairs, epad // (NW * CHUNK))
    return _tc_apply(partials, W, b)


# speedup vs baseline: 2.0263x; 1.4958x over previous
import functools

import jax
import jax.numpy as jnp
from jax import lax
from jax.experimental import pallas as pl
from jax.experimental.pallas import tpu as pltpu
from jax.experimental.pallas import tpu_sc as plsc

N_NODES_K = 10000
D_K = 128
ACC_ROWS = 10240
CHUNK = 128
NC, NS = 2, 16
NW = NC * NS


KDEPTH = 1


def _sc_aggregate(feature, edge_pairs, n0, n1):
    mesh = plsc.VectorSubcoreMesh(core_axis_name="c", subcore_axis_name="s")

    @functools.partial(
        pl.kernel,
        out_type=jax.ShapeDtypeStruct((NC, N_NODES_K, D_K), jnp.float32),
        mesh=mesh,
        scratch_types=[
            pltpu.VMEM((KDEPTH, 2, CHUNK), jnp.int32),
            pltpu.VMEM((CHUNK, D_K), jnp.float32),
            pltpu.VMEM_SHARED((ACC_ROWS, D_K), jnp.float32),
            pltpu.SemaphoreType.DMA((KDEPTH,)),
        ],
    )
    def k(feat_hbm, pairs_hbm, out_hbm, idx_v, rows_v, acc_s, sem_g):
        core = lax.axis_index("c")
        sid = lax.axis_index("s")
        nn = jnp.where(core == 0, n0, n1)
        base = jnp.where(core == 0, sid * n0, NS * n0 + sid * n1)

        @pl.loop(0, 128)
        def _(r):
            @pl.loop(0, D_K, step=16)
            def _(c0):
                rows_v[r, pl.ds(c0, 16)] = jnp.zeros((16,), jnp.float32)

        stripe = ACC_ROWS // NS
        @pl.loop(0, stripe, step=128)
        def _(z):
            pltpu.sync_copy(
                rows_v.at[pl.ds(0, 128)],
                acc_s.at[pl.ds(sid * stripe + z, 128)],
            )

        plsc.subcore_barrier()

        @pl.loop(0, n0)
        def _(j):
            @pl.when(j < nn)
            def _():
                pltpu.sync_copy(pairs_hbm.at[base + j], idx_v.at[0])
                pltpu.async_copy(
                    feat_hbm.at[idx_v.at[0].at[0]], rows_v, sem_g.at[0]
                ).wait()
                pltpu.sync_copy(rows_v, acc_s.at[idx_v.at[0].at[1]], add=True)

        plsc.subcore_barrier()

        @pl.when(sid < NS - 1)
        def _():
            pltpu.sync_copy(
                acc_s.at[pl.ds(sid * stripe, stripe)],
                out_hbm.at[core].at[pl.ds(sid * stripe, stripe)],
            )

        @pl.when(sid == NS - 1)
        def _():
            last = N_NODES_K - (NS - 1) * stripe
            pltpu.sync_copy(
                acc_s.at[pl.ds((NS - 1) * stripe, last)],
                out_hbm.at[core].at[pl.ds((NS - 1) * stripe, last)],
            )

    return k(feature, edge_pairs)


def _tc_body(p_ref, w_ref, b_ref, o_ref):
    agg = p_ref[0] + p_ref[1]
    h = jnp.dot(agg, w_ref[...], preferred_element_type=jnp.float32)
    o_ref[...] = jnp.maximum(h + b_ref[...], 0.0)


def _tc_apply(partials, W, b):
    blk = 2000
    return pl.pallas_call(
        _tc_body,
        grid=(N_NODES_K // blk,),
        in_specs=[
            pl.BlockSpec((NC, blk, D_K), lambda i: (0, i, 0)),
            pl.BlockSpec((D_K, D_K), lambda i: (0, 0)),
            pl.BlockSpec((1, D_K), lambda i: (0, 0)),
        ],
        out_specs=pl.BlockSpec((blk, D_K), lambda i: (i, 0)),
        out_shape=jax.ShapeDtypeStruct((N_NODES_K, D_K), jnp.float32),
    )(partials, W, b.reshape(1, D_K))


def kernel(feature, edge_index, W, b):
    e = edge_index.shape[1]
    quantum = NS * CHUNK
    epad = ((e + quantum - 1) // quantum) * quantum
    pad = epad - e
    src = jnp.concatenate([edge_index[0], jnp.zeros((pad,), jnp.int32)])
    dummy = N_NODES_K + jnp.arange(pad, dtype=jnp.int32) % (
        ACC_ROWS - N_NODES_K
    )
    dst = jnp.concatenate([edge_index[1], dummy])
    pairs = jnp.stack(
        [src.reshape(-1, CHUNK), dst.reshape(-1, CHUNK)], axis=1
    )
    per_pair = epad // (NS * CHUNK)
    n0 = (per_pair * 5 + 4) // 8
    n1 = per_pair - n0
    partials = _sc_aggregate(feature, pairs, n0, n1)
    return _tc_apply(partials, W, b)


# speedup vs baseline: 2.0996x; 1.0362x over previous
import functools

import jax
import jax.numpy as jnp
from jax import lax
from jax.experimental import pallas as pl
from jax.experimental.pallas import tpu as pltpu
from jax.experimental.pallas import tpu_sc as plsc

N_NODES_K = 10000
D_K = 128
ACC_ROWS = 10240
CHUNK = 128
NC, NS = 2, 16
NW = NC * NS


KDEPTH = 1


def _sc_aggregate(feature, edge_pairs, n0, n1):
    mesh = plsc.VectorSubcoreMesh(core_axis_name="c", subcore_axis_name="s")

    @functools.partial(
        pl.kernel,
        out_type=jax.ShapeDtypeStruct((NC, N_NODES_K, D_K), jnp.float32),
        mesh=mesh,
        scratch_types=[
            pltpu.VMEM((KDEPTH, 2, CHUNK), jnp.int32),
            pltpu.VMEM((CHUNK, D_K), jnp.float32),
            pltpu.VMEM_SHARED((ACC_ROWS, D_K), jnp.float32),
            pltpu.SemaphoreType.DMA((KDEPTH,)),
        ],
    )
    def k(feat_hbm, pairs_hbm, out_hbm, idx_v, rows_v, acc_s, sem_g):
        core = lax.axis_index("c")
        sid = lax.axis_index("s")
        nn = jnp.where(core == 0, n0, n1)
        base = jnp.where(core == 0, sid * n0, NS * n0 + sid * n1)

        @pl.loop(0, 128)
        def _(r):
            @pl.loop(0, D_K, step=16)
            def _(c0):
                rows_v[r, pl.ds(c0, 16)] = jnp.zeros((16,), jnp.float32)

        stripe = ACC_ROWS // NS
        @pl.loop(0, stripe, step=128)
        def _(z):
            pltpu.sync_copy(
                rows_v.at[pl.ds(0, 128)],
                acc_s.at[pl.ds(sid * stripe + z, 128)],
            )

        plsc.subcore_barrier()

        @pl.loop(0, n0)
        def _(j):
            @pl.when(j < nn)
            def _():
                pltpu.sync_copy(pairs_hbm.at[base + j], idx_v.at[0])
                pltpu.async_copy(
                    feat_hbm.at[idx_v.at[0].at[0]], rows_v, sem_g.at[0]
                ).wait()
                pltpu.sync_copy(rows_v, acc_s.at[idx_v.at[0].at[1]], add=True)

        plsc.subcore_barrier()

        @pl.when(sid < NS - 1)
        def _():
            pltpu.sync_copy(
                acc_s.at[pl.ds(sid * stripe, stripe)],
                out_hbm.at[core].at[pl.ds(sid * stripe, stripe)],
            )

        @pl.when(sid == NS - 1)
        def _():
            last = N_NODES_K - (NS - 1) * stripe
            pltpu.sync_copy(
                acc_s.at[pl.ds((NS - 1) * stripe, last)],
                out_hbm.at[core].at[pl.ds((NS - 1) * stripe, last)],
            )

    return k(feature, edge_pairs)


def _tc_body(p_ref, w_ref, b_ref, o_ref):
    agg = p_ref[0] + p_ref[1]
    h = jnp.dot(agg, w_ref[...], preferred_element_type=jnp.float32)
    o_ref[...] = jnp.maximum(h + b_ref[...], 0.0)


def _tc_apply(partials, W, b):
    blk = 2000
    return pl.pallas_call(
        _tc_body,
        grid=(N_NODES_K // blk,),
        in_specs=[
            pl.BlockSpec((NC, blk, D_K), lambda i: (0, i, 0)),
            pl.BlockSpec((D_K, D_K), lambda i: (0, 0)),
            pl.BlockSpec((1, D_K), lambda i: (0, 0)),
        ],
        out_specs=pl.BlockSpec((blk, D_K), lambda i: (i, 0)),
        out_shape=jax.ShapeDtypeStruct((N_NODES_K, D_K), jnp.float32),
    )(partials, W, b.reshape(1, D_K))


def kernel(feature, edge_index, W, b):
    e = edge_index.shape[1]
    quantum = NS * CHUNK
    epad = ((e + quantum - 1) // quantum) * quantum
    pad = epad - e
    src = jnp.concatenate([edge_index[0], jnp.zeros((pad,), jnp.int32)])
    dummy = N_NODES_K + jnp.arange(pad, dtype=jnp.int32) % (
        ACC_ROWS - N_NODES_K
    )
    dst = jnp.concatenate([edge_index[1], dummy])
    pairs = jnp.stack(
        [src.reshape(-1, CHUNK), dst.reshape(-1, CHUNK)], axis=1
    )
    per_pair = epad // (NS * CHUNK)
    n0 = (per_pair * 3) // 5
    n1 = per_pair - n0
    partials = _sc_aggregate(feature, pairs, n0, n1)
    return _tc_apply(partials, W, b)


# speedup vs baseline: 2.4279x; 1.1563x over previous
import functools

import jax
import jax.numpy as jnp
from jax import lax
from jax.experimental import pallas as pl
from jax.experimental.pallas import tpu as pltpu
from jax.experimental.pallas import tpu_sc as plsc

N_NODES_K = 10000
D_K = 128
ACC_ROWS = 10240
CHUNK = 128
NC, NS = 2, 16
NW = NC * NS


KDEPTH = 1


def _sc_aggregate(feature, edge_pairs, n0, n1):
    mesh = plsc.VectorSubcoreMesh(core_axis_name="c", subcore_axis_name="s")

    @functools.partial(
        pl.kernel,
        out_type=jax.ShapeDtypeStruct((NC, N_NODES_K, D_K), jnp.float32),
        mesh=mesh,
        scratch_types=[
            pltpu.VMEM((2, 2, CHUNK), jnp.int32),
            pltpu.VMEM((CHUNK, D_K), jnp.float32),
            pltpu.VMEM_SHARED((ACC_ROWS, D_K), jnp.float32),
            pltpu.SemaphoreType.DMA((1,)),
            pltpu.SemaphoreType.DMA((2,)),
        ],
    )
    def k(feat_hbm, pairs_hbm, out_hbm, idx_v, rows_v, acc_s, sem_g, sem_i):
        core = lax.axis_index("c")
        sid = lax.axis_index("s")
        nn = jnp.where(core == 0, n0, n1)
        base = jnp.where(core == 0, sid * n0, NS * n0 + sid * n1)

        @pl.loop(0, 128)
        def _(r):
            @pl.loop(0, D_K, step=16)
            def _(c0):
                rows_v[r, pl.ds(c0, 16)] = jnp.zeros((16,), jnp.float32)

        stripe = ACC_ROWS // NS
        @pl.loop(0, stripe, step=128)
        def _(z):
            pltpu.sync_copy(
                rows_v.at[pl.ds(0, 128)],
                acc_s.at[pl.ds(sid * stripe + z, 128)],
            )

        plsc.subcore_barrier()

        pltpu.sync_copy(pairs_hbm.at[base], idx_v.at[0])

        @pl.loop(0, n0 + (n0 % 2), step=2)
        def _(j):
            for t in range(2):
                cur = idx_v.at[t]
                nxt = idx_v.at[1 - t]

                @pl.when(j + t < nn)
                def _():
                    @pl.when(j + t + 1 < nn)
                    def _():
                        pltpu.async_copy(
                            pairs_hbm.at[base + j + t + 1], nxt, sem_i.at[1 - t]
                        )

                    pltpu.async_copy(
                        feat_hbm.at[cur.at[0]], rows_v, sem_g.at[0]
                    ).wait()
                    pltpu.sync_copy(rows_v, acc_s.at[cur.at[1]], add=True)

                    @pl.when(j + t + 1 < nn)
                    def _():
                        pltpu.make_async_copy(
                            pairs_hbm.at[base], nxt, sem_i.at[1 - t]
                        ).wait()

        plsc.subcore_barrier()

        @pl.when(sid < NS - 1)
        def _():
            pltpu.sync_copy(
                acc_s.at[pl.ds(sid * stripe, stripe)],
                out_hbm.at[core].at[pl.ds(sid * stripe, stripe)],
            )

        @pl.when(sid == NS - 1)
        def _():
            last = N_NODES_K - (NS - 1) * stripe
            pltpu.sync_copy(
                acc_s.at[pl.ds((NS - 1) * stripe, last)],
                out_hbm.at[core].at[pl.ds((NS - 1) * stripe, last)],
            )

    return k(feature, edge_pairs)


def _tc_body(p_ref, w_ref, b_ref, o_ref):
    agg = p_ref[0] + p_ref[1]
    h = jnp.dot(agg, w_ref[...], preferred_element_type=jnp.float32)
    o_ref[...] = jnp.maximum(h + b_ref[...], 0.0)


def _tc_apply(partials, W, b):
    blk = 2000
    return pl.pallas_call(
        _tc_body,
        grid=(N_NODES_K // blk,),
        in_specs=[
            pl.BlockSpec((NC, blk, D_K), lambda i: (0, i, 0)),
            pl.BlockSpec((D_K, D_K), lambda i: (0, 0)),
            pl.BlockSpec((1, D_K), lambda i: (0, 0)),
        ],
        out_specs=pl.BlockSpec((blk, D_K), lambda i: (i, 0)),
        out_shape=jax.ShapeDtypeStruct((N_NODES_K, D_K), jnp.float32),
    )(partials, W, b.reshape(1, D_K))


def kernel(feature, edge_index, W, b):
    e = edge_index.shape[1]
    quantum = NS * CHUNK
    epad = ((e + quantum - 1) // quantum) * quantum
    pad = epad - e
    src = jnp.concatenate([edge_index[0], jnp.zeros((pad,), jnp.int32)])
    dummy = N_NODES_K + jnp.arange(pad, dtype=jnp.int32) % (
        ACC_ROWS - N_NODES_K
    )
    dst = jnp.concatenate([edge_index[1], dummy])
    pairs = jnp.stack(
        [src.reshape(-1, CHUNK), dst.reshape(-1, CHUNK)], axis=1
    )
    per_pair = epad // (NS * CHUNK)
    n0 = (per_pair * 3) // 5
    n1 = per_pair - n0
    partials = _sc_aggregate(feature, pairs, n0, n1)
    return _tc_apply(partials, W, b)


# speedup vs baseline: 2.7056x; 1.1144x over previous
import functools

import jax
import jax.numpy as jnp
from jax import lax
from jax.experimental import pallas as pl
from jax.experimental.pallas import tpu as pltpu
from jax.experimental.pallas import tpu_sc as plsc

N_NODES_K = 10000
D_K = 128
ACC_ROWS = 10240
CHUNK = 128
NC, NS = 2, 16


def _sc_aggregate(feature, edges, t0_chunks, t1_chunks):
    mesh = plsc.VectorSubcoreMesh(core_axis_name="c", subcore_axis_name="s")

    q0, r0 = divmod(t0_chunks, NS)
    q1, r1 = divmod(t1_chunks, NS)
    nmax = max(q0 + (1 if r0 else 0), q1 + (1 if r1 else 0))

    @functools.partial(
        pl.kernel,
        out_type=jax.ShapeDtypeStruct((NC, N_NODES_K, D_K), jnp.float32),
        mesh=mesh,
        scratch_types=[
            pltpu.VMEM((2, 2, CHUNK), jnp.int32),
            pltpu.VMEM((CHUNK, D_K), jnp.float32),
            pltpu.VMEM_SHARED((ACC_ROWS, D_K), jnp.float32),
            pltpu.SemaphoreType.DMA((1,)),
            pltpu.SemaphoreType.DMA((2,)),
        ],
    )
    def k(feat_hbm, edges_hbm, out_hbm, idx_v, rows_v, acc_s, sem_g, sem_i):
        core = lax.axis_index("c")
        sid = lax.axis_index("s")
        nn = jnp.where(
            core == 0,
            jnp.where(sid < r0, q0 + 1, q0),
            jnp.where(sid < r1, q1 + 1, q1),
        )
        base = jnp.where(
            core == 0,
            sid * q0 + jnp.minimum(sid, r0),
            t0_chunks + sid * q1 + jnp.minimum(sid, r1),
        )

        def load_idx(chunk, slot, sem):
            off = (base + chunk) * CHUNK
            pltpu.async_copy(
                edges_hbm.at[0].at[pl.ds(off, CHUNK)], slot.at[0], sem
            )
            pltpu.async_copy(
                edges_hbm.at[1].at[pl.ds(off, CHUNK)], slot.at[1], sem
            )

        def wait_idx(slot, sem):
            pltpu.make_async_copy(
                edges_hbm.at[0].at[pl.ds(0, CHUNK)], slot.at[0], sem
            ).wait()
            pltpu.make_async_copy(
                edges_hbm.at[1].at[pl.ds(0, CHUNK)], slot.at[1], sem
            ).wait()

        @pl.loop(0, 128)
        def _(r):
            @pl.loop(0, D_K, step=16)
            def _(c0):
                rows_v[r, pl.ds(c0, 16)] = jnp.zeros((16,), jnp.float32)

        stripe = ACC_ROWS // NS
        @pl.loop(0, stripe, step=128)
        def _(z):
            pltpu.sync_copy(
                rows_v.at[pl.ds(0, 128)],
                acc_s.at[pl.ds(sid * stripe + z, 128)],
            )

        plsc.subcore_barrier()

        load_idx(0, idx_v.at[0], sem_i.at[0])
        wait_idx(idx_v.at[0], sem_i.at[0])

        @pl.loop(0, nmax + (nmax % 2), step=2)
        def _(j):
            for t in range(2):
                cur = idx_v.at[t]
                nxt = idx_v.at[1 - t]

                @pl.when(j + t < nn)
                def _():
                    @pl.when(j + t + 1 < nn)
                    def _():
                        load_idx(j + t + 1, nxt, sem_i.at[1 - t])

                    pltpu.async_copy(
                        feat_hbm.at[cur.at[0]], rows_v, sem_g.at[0]
                    ).wait()
                    pltpu.sync_copy(rows_v, acc_s.at[cur.at[1]], add=True)

                    @pl.when(j + t + 1 < nn)
                    def _():
                        wait_idx(nxt, sem_i.at[1 - t])

        plsc.subcore_barrier()

        @pl.when(sid < NS - 1)
        def _():
            pltpu.sync_copy(
                acc_s.at[pl.ds(sid * stripe, stripe)],
                out_hbm.at[core].at[pl.ds(sid * stripe, stripe)],
            )

        @pl.when(sid == NS - 1)
        def _():
            last = N_NODES_K - (NS - 1) * stripe
            pltpu.sync_copy(
                acc_s.at[pl.ds((NS - 1) * stripe, last)],
                out_hbm.at[core].at[pl.ds((NS - 1) * stripe, last)],
            )

    return k(feature, edges)


def _tc_body(p_ref, w_ref, b_ref, o_ref):
    agg = p_ref[0] + p_ref[1]
    h = jnp.dot(agg, w_ref[...], preferred_element_type=jnp.float32)
    o_ref[...] = jnp.maximum(h + b_ref[...], 0.0)


def _tc_apply(partials, W, b):
    blk = 2000
    return pl.pallas_call(
        _tc_body,
        grid=(N_NODES_K // blk,),
        in_specs=[
            pl.BlockSpec((NC, blk, D_K), lambda i: (0, i, 0)),
            pl.BlockSpec((D_K, D_K), lambda i: (0, 0)),
            pl.BlockSpec((1, D_K), lambda i: (0, 0)),
        ],
        out_specs=pl.BlockSpec((blk, D_K), lambda i: (i, 0)),
        out_shape=jax.ShapeDtypeStruct((N_NODES_K, D_K), jnp.float32),
    )(partials, W, b.reshape(1, D_K))


def kernel(feature, edge_index, W, b):
    e = edge_index.shape[1]
    pad = (-e) % CHUNK
    if pad:
        dummy = N_NODES_K + jnp.arange(pad, dtype=jnp.int32) % (
            ACC_ROWS - N_NODES_K
        )
        pad_block = jnp.stack([jnp.zeros((pad,), jnp.int32), dummy])
        edge_index = jnp.concatenate([edge_index, pad_block], axis=1)
    t = (e + pad) // CHUNK
    t0 = (t * 3) // 5
    t1 = t - t0
    partials = _sc_aggregate(feature, edge_index, t0, t1)
    return _tc_apply(partials, W, b)


# speedup vs baseline: 3.0808x; 1.1386x over previous
import functools

import jax
import jax.numpy as jnp
from jax import lax
from jax.experimental import pallas as pl
from jax.experimental.pallas import tpu as pltpu
from jax.experimental.pallas import tpu_sc as plsc

N_NODES_K = 10000
D_K = 128
ACC_ROWS = 10240
CHUNK = 128
NC, NS = 2, 16


def _sc_aggregate(feature, edges, t0_chunks, t1_chunks):
    mesh = plsc.VectorSubcoreMesh(core_axis_name="c", subcore_axis_name="s")

    q0, r0 = divmod(t0_chunks, NS)
    q1, r1 = divmod(t1_chunks, NS)
    nmax = max(q0 + (1 if r0 else 0), q1 + (1 if r1 else 0))

    @functools.partial(
        pl.kernel,
        out_type=jax.ShapeDtypeStruct((NC, N_NODES_K, D_K), jnp.float32),
        mesh=mesh,
        scratch_types=[
            pltpu.VMEM((2, 2, CHUNK), jnp.int32),
            pltpu.VMEM((CHUNK, D_K), jnp.float32),
            pltpu.VMEM_SHARED((ACC_ROWS, D_K), jnp.float32),
            pltpu.SemaphoreType.DMA((1,)),
            pltpu.SemaphoreType.DMA((2,)),
        ],
    )
    def k(feat_hbm, edges_hbm, out_hbm, idx_v, rows_v, acc_s, sem_g, sem_i):
        core = lax.axis_index("c")
        sid = lax.axis_index("s")
        nn = jnp.where(
            core == 0,
            jnp.where(sid < r0, q0 + 1, q0),
            jnp.where(sid < r1, q1 + 1, q1),
        )
        base = jnp.where(
            core == 0,
            sid * q0 + jnp.minimum(sid, r0),
            t0_chunks + sid * q1 + jnp.minimum(sid, r1),
        )

        def load_idx(chunk, slot, sem):
            off = (base + chunk) * CHUNK
            pltpu.async_copy(
                edges_hbm.at[0].at[pl.ds(off, CHUNK)], slot.at[0], sem
            )
            pltpu.async_copy(
                edges_hbm.at[1].at[pl.ds(off, CHUNK)], slot.at[1], sem
            )

        def wait_idx(slot, sem):
            pltpu.make_async_copy(
                edges_hbm.at[0].at[pl.ds(0, CHUNK)], slot.at[0], sem
            ).wait()
            pltpu.make_async_copy(
                edges_hbm.at[1].at[pl.ds(0, CHUNK)], slot.at[1], sem
            ).wait()

        @pl.loop(0, 128)
        def _(r):
            @pl.loop(0, D_K, step=16)
            def _(c0):
                rows_v[r, pl.ds(c0, 16)] = jnp.zeros((16,), jnp.float32)

        stripe = ACC_ROWS // NS
        @pl.loop(0, stripe, step=128)
        def _(z):
            pltpu.sync_copy(
                rows_v.at[pl.ds(0, 128)],
                acc_s.at[pl.ds(sid * stripe + z, 128)],
            )

        plsc.subcore_barrier()

        load_idx(0, idx_v.at[0], sem_i.at[0])
        wait_idx(idx_v.at[0], sem_i.at[0])

        @pl.loop(0, nmax + (nmax % 2), step=2)
        def _(j):
            for t in range(2):
                cur = idx_v.at[t]
                nxt = idx_v.at[1 - t]

                @pl.when(j + t < nn)
                def _():
                    @pl.when(j + t + 1 < nn)
                    def _():
                        load_idx(j + t + 1, nxt, sem_i.at[1 - t])

                    pltpu.async_copy(
                        feat_hbm.at[cur.at[0]], rows_v, sem_g.at[0]
                    ).wait()
                    pltpu.sync_copy(rows_v, acc_s.at[cur.at[1]], add=True)

                    @pl.when(j + t + 1 < nn)
                    def _():
                        wait_idx(nxt, sem_i.at[1 - t])

        plsc.subcore_barrier()

        @pl.when(sid < NS - 1)
        def _():
            pltpu.sync_copy(
                acc_s.at[pl.ds(sid * stripe, stripe)],
                out_hbm.at[core].at[pl.ds(sid * stripe, stripe)],
            )

        @pl.when(sid == NS - 1)
        def _():
            last = N_NODES_K - (NS - 1) * stripe
            pltpu.sync_copy(
                acc_s.at[pl.ds((NS - 1) * stripe, last)],
                out_hbm.at[core].at[pl.ds((NS - 1) * stripe, last)],
            )

    return k(feature, edges)


def _tc_body(p_ref, w_ref, b_ref, o_ref):
    agg = p_ref[0] + p_ref[1]
    h = jnp.dot(agg, w_ref[...], preferred_element_type=jnp.float32)
    o_ref[...] = jnp.maximum(h + b_ref[...], 0.0)


def _tc_apply(partials, W, b):
    blk = 2000
    return pl.pallas_call(
        _tc_body,
        grid=(N_NODES_K // blk,),
        in_specs=[
            pl.BlockSpec((NC, blk, D_K), lambda i: (0, i, 0)),
            pl.BlockSpec((D_K, D_K), lambda i: (0, 0)),
            pl.BlockSpec((1, D_K), lambda i: (0, 0)),
        ],
        out_specs=pl.BlockSpec((blk, D_K), lambda i: (i, 0)),
        out_shape=jax.ShapeDtypeStruct((N_NODES_K, D_K), jnp.float32),
    )(partials, W, b.reshape(1, D_K))


def kernel(feature, edge_index, W, b):
    e = edge_index.shape[1]
    pad = (-e) % CHUNK
    if pad:
        dummy = N_NODES_K + jnp.arange(pad, dtype=jnp.int32) % (
            ACC_ROWS - N_NODES_K
        )
        pad_block = jnp.stack([jnp.zeros((pad,), jnp.int32), dummy])
        edge_index = jnp.concatenate([edge_index, pad_block], axis=1)
    t = (e + pad) // CHUNK
    t0 = (t * 51) // 100
    t1 = t - t0
    partials = _sc_aggregate(feature, edge_index, t0, t1)
    return _tc_apply(partials, W, b)
